# baseline (speedup 1.0000x reference)
import jax
import jax.numpy as jnp
from jax.experimental import pallas as pl


def _mlp_body(h0_ref, w1_ref, b1_ref, w2_ref, b2_ref, w3_ref, b3_ref, out_ref):
    h = jax.nn.relu(h0_ref[...] @ w1_ref[...] + b1_ref[...])
    h = jax.nn.relu(h @ w2_ref[...] + b2_ref[...])
    out_ref[...] = h @ w3_ref[...] + b3_ref[...]


def kernel(pred_feat, plan_feat0, plan_feat1, src_pred, dst_and, src_and, dst_or, map0, map1, src_plan1, dst_plan0, bn_gamma, bn_beta, W_pred, b_pred, W_plan, b_plan, W1, b1, W2, b2, W3, b3):
    N_AND = 50000
    N_OR = 25000
    N_PLAN0 = 25000
    mu = pred_feat.mean(axis=0)
    var = pred_feat.var(axis=0)
    xn = (pred_feat - mu) / jnp.sqrt(var + 1e-5) * bn_gamma + bn_beta
    pred_enc = jax.nn.relu(xn @ W_pred + b_pred)
    and_h = jax.ops.segment_min(pred_enc[src_pred], dst_and, num_segments=N_AND)
    and_h = jnp.where(jnp.isfinite(and_h), and_h, 0.0)
    or_h = jax.ops.segment_max(and_h[src_and], dst_or, num_segments=N_OR)
    or_h = jnp.where(jnp.isfinite(or_h), or_h, 0.0)
    pph0 = or_h[map0]
    pph1 = pred_enc[map1]
    enc0 = jnp.concatenate([jax.nn.relu(plan_feat0 @ W_plan + b_plan), pph0], axis=1)
    enc1 = jnp.concatenate([jax.nn.relu(plan_feat1 @ W_plan + b_plan), pph1], axis=1)
    agg = jax.ops.segment_sum(enc1[src_plan1], dst_plan0, num_segments=N_PLAN0)
    h0 = enc0 + agg

    RB = 1000
    out = pl.pallas_call(
        _mlp_body,
        grid=(N_PLAN0 // RB,),
        in_specs=[
            pl.BlockSpec((RB, 128), lambda i: (i, 0)),
            pl.BlockSpec((128, 128), lambda i: (0, 0)),
            pl.BlockSpec((128,), lambda i: (0,)),
            pl.BlockSpec((128, 128), lambda i: (0, 0)),
            pl.BlockSpec((128,), lambda i: (0,)),
            pl.BlockSpec((128, 1), lambda i: (0, 0)),
            pl.BlockSpec((1,), lambda i: (0,)),
        ],
        out_specs=pl.BlockSpec((RB, 1), lambda i: (i, 0)),
        out_shape=jax.ShapeDtypeStruct((N_PLAN0, 1), jnp.float32),
    )(h0, W1, b1, W2, b2, W3, b3)
    return out
